# baseline (speedup 1.0000x reference)
import functools

import jax
import jax.numpy as jnp
from jax.experimental import pallas as pl
from jax.experimental.pallas import tpu as pltpu

B, S, H = 1, 2048, 1024
I = 768
E = 8
K = 2
T = B * S
BT = 256


def _moe_body(x_ref, wr_ref, wup_ref, bup_ref, wdn_ref, bdn_ref,
              out_ref, logits_ref, counts_ref, cw_ref):
    t = pl.program_id(0)
    e = pl.program_id(1)
    x = x_ref[...]

    @pl.when(e == 0)
    def _router():
        logits = jnp.dot(x, wr_ref[...], preferred_element_type=jnp.float32)
        logits_ref[...] = logits
        m = jnp.max(logits, axis=-1, keepdims=True)
        ex = jnp.exp(logits - m)
        rw = ex / jnp.sum(ex, axis=-1, keepdims=True)
        idx = jax.lax.broadcasted_iota(jnp.int32, (BT, E), 1)
        m1 = jnp.max(rw, axis=-1, keepdims=True)
        a1 = jnp.min(jnp.where(rw == m1, idx, E), axis=-1, keepdims=True)
        mask1 = idx == a1
        rw2 = jnp.where(mask1, -1.0, rw)
        m2 = jnp.max(rw2, axis=-1, keepdims=True)
        a2 = jnp.min(jnp.where(rw2 == m2, idx, E), axis=-1, keepdims=True)
        mask2 = idx == a2
        denom = m1 + m2
        cw = jnp.where(mask1, m1, jnp.where(mask2, m2, 0.0)) / denom
        cw_ref[...] = cw
        c = jnp.sum((mask1 | mask2).astype(jnp.float32), axis=0, keepdims=True)

        @pl.when(t == 0)
        def _():
            counts_ref[...] = c

        @pl.when(t != 0)
        def _():
            counts_ref[...] += c

    eidx = jax.lax.broadcasted_iota(jnp.int32, (BT, E), 1)
    w_e = jnp.sum(jnp.where(eidx == e, cw_ref[...], 0.0), axis=-1,
                  keepdims=True)

    up = jnp.dot(x, wup_ref[0], preferred_element_type=jnp.float32)
    up = up + bup_ref[0]
    act = up * jax.nn.sigmoid(up)
    dn = jnp.dot(act, wdn_ref[0], preferred_element_type=jnp.float32)
    dn = dn + bdn_ref[0]
    contrib = dn * w_e

    @pl.when(e == 0)
    def _():
        out_ref[...] = contrib

    @pl.when(e != 0)
    def _():
        out_ref[...] += contrib


@jax.jit
def _moe(xf, W_router, W_up, b_up, W_down, b_down):
    grid = (T // BT, E)
    out, logits, counts = pl.pallas_call(
        _moe_body,
        grid=grid,
        in_specs=[
            pl.BlockSpec((BT, H), lambda t, e: (t, 0)),
            pl.BlockSpec((H, E), lambda t, e: (0, 0)),
            pl.BlockSpec((1, H, I), lambda t, e: (e, 0, 0)),
            pl.BlockSpec((1, 1, I), lambda t, e: (e, 0, 0)),
            pl.BlockSpec((1, I, H), lambda t, e: (e, 0, 0)),
            pl.BlockSpec((1, 1, H), lambda t, e: (e, 0, 0)),
        ],
        out_specs=[
            pl.BlockSpec((BT, H), lambda t, e: (t, 0)),
            pl.BlockSpec((BT, E), lambda t, e: (t, 0)),
            pl.BlockSpec((1, E), lambda t, e: (0, 0)),
        ],
        out_shape=[
            jax.ShapeDtypeStruct((T, H), jnp.float32),
            jax.ShapeDtypeStruct((T, E), jnp.float32),
            jax.ShapeDtypeStruct((1, E), jnp.float32),
        ],
        scratch_shapes=[pltpu.VMEM((BT, E), jnp.float32)],
    )(xf, W_router, W_up, b_up.reshape(E, 1, I), W_down, b_down.reshape(E, 1, H))
    return out, logits, counts


def kernel(hidden_states, W_router, W_up, b_up, W_down, b_down):
    xf = hidden_states.reshape(T, H)
    out, logits, counts = _moe(xf, W_router, W_up, b_up, W_down, b_down)
    usage = counts[0] * (E / (T * K))
    s = jnp.sum(usage)
    aux_loss = s * s / (E * E)
    return out.reshape(B, S, H), logits, aux_loss


# speedup vs baseline: 1.0012x; 1.0012x over previous
import functools

import jax
import jax.numpy as jnp
from jax.experimental import pallas as pl
from jax.experimental.pallas import tpu as pltpu

B, S, H = 1, 2048, 1024
I = 768
E = 8
K = 2
T = B * S
BT = 256


def _moe_body(x_ref, wr_ref, wup_ref, bup_ref, wdn_ref, bdn_ref,
              out_ref, logits_ref, counts_ref, cw_ref):
    t = pl.program_id(0)
    e = pl.program_id(1)
    x = x_ref[...]

    @pl.when(e == 0)
    def _router():
        logits = jnp.dot(x, wr_ref[...], preferred_element_type=jnp.float32)
        logits_ref[...] = logits
        m = jnp.max(logits, axis=-1, keepdims=True)
        ex = jnp.exp(logits - m)
        rw = ex / jnp.sum(ex, axis=-1, keepdims=True)
        idx = jax.lax.broadcasted_iota(jnp.int32, (BT, E), 1)
        m1 = jnp.max(rw, axis=-1, keepdims=True)
        a1 = jnp.min(jnp.where(rw == m1, idx, E), axis=-1, keepdims=True)
        mask1 = idx == a1
        rw2 = jnp.where(mask1, -1.0, rw)
        m2 = jnp.max(rw2, axis=-1, keepdims=True)
        a2 = jnp.min(jnp.where(rw2 == m2, idx, E), axis=-1, keepdims=True)
        mask2 = idx == a2
        denom = m1 + m2
        cw = jnp.where(mask1, m1, jnp.where(mask2, m2, 0.0)) / denom
        cw_ref[...] = cw
        c = jnp.sum((mask1 | mask2).astype(jnp.float32), axis=0, keepdims=True)

        @pl.when(t == 0)
        def _():
            counts_ref[...] = c

        @pl.when(t != 0)
        def _():
            counts_ref[...] += c

    eidx = jax.lax.broadcasted_iota(jnp.int32, (BT, E), 1)
    w_e = jnp.sum(jnp.where(eidx == e, cw_ref[...], 0.0), axis=-1,
                  keepdims=True)

    up = jnp.dot(x.astype(jnp.bfloat16), wup_ref[0].astype(jnp.bfloat16),
                 preferred_element_type=jnp.float32)
    up = up + bup_ref[0]
    act = up * jax.nn.sigmoid(up)
    dn = jnp.dot(act.astype(jnp.bfloat16), wdn_ref[0].astype(jnp.bfloat16),
                 preferred_element_type=jnp.float32)
    dn = dn + bdn_ref[0]
    contrib = dn * w_e

    @pl.when(e == 0)
    def _():
        out_ref[...] = contrib

    @pl.when(e != 0)
    def _():
        out_ref[...] += contrib


@jax.jit
def _moe(xf, W_router, W_up, b_up, W_down, b_down):
    grid = (T // BT, E)
    out, logits, counts = pl.pallas_call(
        _moe_body,
        grid=grid,
        in_specs=[
            pl.BlockSpec((BT, H), lambda t, e: (t, 0)),
            pl.BlockSpec((H, E), lambda t, e: (0, 0)),
            pl.BlockSpec((1, H, I), lambda t, e: (e, 0, 0)),
            pl.BlockSpec((1, 1, I), lambda t, e: (e, 0, 0)),
            pl.BlockSpec((1, I, H), lambda t, e: (e, 0, 0)),
            pl.BlockSpec((1, 1, H), lambda t, e: (e, 0, 0)),
        ],
        out_specs=[
            pl.BlockSpec((BT, H), lambda t, e: (t, 0)),
            pl.BlockSpec((BT, E), lambda t, e: (t, 0)),
            pl.BlockSpec((1, E), lambda t, e: (0, 0)),
        ],
        out_shape=[
            jax.ShapeDtypeStruct((T, H), jnp.float32),
            jax.ShapeDtypeStruct((T, E), jnp.float32),
            jax.ShapeDtypeStruct((1, E), jnp.float32),
        ],
        scratch_shapes=[pltpu.VMEM((BT, E), jnp.float32)],
    )(xf, W_router, W_up, b_up.reshape(E, 1, I), W_down, b_down.reshape(E, 1, H))
    return out, logits, counts


def kernel(hidden_states, W_router, W_up, b_up, W_down, b_down):
    xf = hidden_states.reshape(T, H)
    out, logits, counts = _moe(xf, W_router, W_up, b_up, W_down, b_down)
    usage = counts[0] * (E / (T * K))
    s = jnp.sum(usage)
    aux_loss = s * s / (E * E)
    return out.reshape(B, S, H), logits, aux_loss


# speedup vs baseline: 1.5950x; 1.5930x over previous
import functools

import jax
import jax.numpy as jnp
from jax.experimental import pallas as pl
from jax.experimental.pallas import tpu as pltpu

B, S, H = 1, 2048, 1024
I = 768
E = 8
K = 2
T = B * S
BT = 2048


def _moe_body(x_ref, wr_ref, wup_ref, bup_ref, wdn_ref, bdn_ref,
              out_ref, logits_ref, counts_ref, cw_ref):
    t = pl.program_id(0)
    e = pl.program_id(1)
    x = x_ref[...]

    @pl.when(e == 0)
    def _router():
        logits = jnp.dot(x, wr_ref[...], preferred_element_type=jnp.float32)
        logits_ref[...] = logits
        m = jnp.max(logits, axis=-1, keepdims=True)
        ex = jnp.exp(logits - m)
        rw = ex / jnp.sum(ex, axis=-1, keepdims=True)
        idx = jax.lax.broadcasted_iota(jnp.int32, (BT, E), 1)
        m1 = jnp.max(rw, axis=-1, keepdims=True)
        a1 = jnp.min(jnp.where(rw == m1, idx, E), axis=-1, keepdims=True)
        mask1 = idx == a1
        rw2 = jnp.where(mask1, -1.0, rw)
        m2 = jnp.max(rw2, axis=-1, keepdims=True)
        a2 = jnp.min(jnp.where(rw2 == m2, idx, E), axis=-1, keepdims=True)
        mask2 = idx == a2
        denom = m1 + m2
        cw = jnp.where(mask1, m1, jnp.where(mask2, m2, 0.0)) / denom
        cw_ref[...] = cw
        c = jnp.sum((mask1 | mask2).astype(jnp.float32), axis=0, keepdims=True)

        @pl.when(t == 0)
        def _():
            counts_ref[...] = c

        @pl.when(t != 0)
        def _():
            counts_ref[...] += c

    eidx = jax.lax.broadcasted_iota(jnp.int32, (BT, E), 1)
    w_e = jnp.sum(jnp.where(eidx == e, cw_ref[...], 0.0), axis=-1,
                  keepdims=True)

    up = jnp.dot(x.astype(jnp.bfloat16), wup_ref[0].astype(jnp.bfloat16),
                 preferred_element_type=jnp.float32)
    up = up + bup_ref[0]
    act = up * jax.nn.sigmoid(up)
    dn = jnp.dot(act.astype(jnp.bfloat16), wdn_ref[0].astype(jnp.bfloat16),
                 preferred_element_type=jnp.float32)
    dn = dn + bdn_ref[0]
    contrib = dn * w_e

    @pl.when(e == 0)
    def _():
        out_ref[...] = contrib

    @pl.when(e != 0)
    def _():
        out_ref[...] += contrib


@jax.jit
def _moe(xf, W_router, W_up, b_up, W_down, b_down):
    grid = (T // BT, E)
    out, logits, counts = pl.pallas_call(
        _moe_body,
        grid=grid,
        in_specs=[
            pl.BlockSpec((BT, H), lambda t, e: (t, 0)),
            pl.BlockSpec((H, E), lambda t, e: (0, 0)),
            pl.BlockSpec((1, H, I), lambda t, e: (e, 0, 0)),
            pl.BlockSpec((1, 1, I), lambda t, e: (e, 0, 0)),
            pl.BlockSpec((1, I, H), lambda t, e: (e, 0, 0)),
            pl.BlockSpec((1, 1, H), lambda t, e: (e, 0, 0)),
        ],
        out_specs=[
            pl.BlockSpec((BT, H), lambda t, e: (t, 0)),
            pl.BlockSpec((BT, E), lambda t, e: (t, 0)),
            pl.BlockSpec((1, E), lambda t, e: (0, 0)),
        ],
        out_shape=[
            jax.ShapeDtypeStruct((T, H), jnp.float32),
            jax.ShapeDtypeStruct((T, E), jnp.float32),
            jax.ShapeDtypeStruct((1, E), jnp.float32),
        ],
        scratch_shapes=[pltpu.VMEM((BT, E), jnp.float32)],
    )(xf, W_router, W_up, b_up.reshape(E, 1, I), W_down, b_down.reshape(E, 1, H))
    return out, logits, counts


def kernel(hidden_states, W_router, W_up, b_up, W_down, b_down):
    xf = hidden_states.reshape(T, H)
    out, logits, counts = _moe(xf, W_router, W_up, b_up, W_down, b_down)
    usage = counts[0] * (E / (T * K))
    s = jnp.sum(usage)
    aux_loss = s * s / (E * E)
    return out.reshape(B, S, H), logits, aux_loss
